# Initial kernel scaffold; baseline (speedup 1.0000x reference)
import functools

import jax
import jax.numpy as jnp
from jax import lax
from jax.experimental import pallas as pl
from jax.experimental.pallas import tpu as pltpu
from jax.experimental.pallas import tpu_sc as plsc

N = 10000
NSPLIT = 5000
NH = 5008
NPAD = 2 * NH
E = 160000
H = 8
C = 32
HC = H * C
NC = 2
NS = 16
NW = NC * NS


def _mesh():
    return plsc.VectorSubcoreMesh(core_axis_name="c", subcore_axis_name="s")


def _mm2(x, w1, w2):
    n, k = x.shape
    mo = w1.shape[1]
    br = 2504

    def body(x_ref, w1_ref, w2_ref, o1_ref, o2_ref):
        xb = x_ref[...]
        o1_ref[...] = jnp.dot(xb, w1_ref[...], preferred_element_type=jnp.float32)
        o2_ref[...] = jnp.dot(xb, w2_ref[...], preferred_element_type=jnp.float32)

    return pl.pallas_call(
        body,
        grid=(n // br,),
        in_specs=[
            pl.BlockSpec((br, k), lambda i: (i, 0)),
            pl.BlockSpec((k, mo), lambda i: (0, 0)),
            pl.BlockSpec((k, mo), lambda i: (0, 0)),
        ],
        out_specs=[
            pl.BlockSpec((br, mo), lambda i: (i, 0)),
            pl.BlockSpec((br, mo), lambda i: (i, 0)),
        ],
        out_shape=[jax.ShapeDtypeStruct((n, mo), jnp.float32),
                   jax.ShapeDtypeStruct((n, mo), jnp.float32)],
    )(x, w1, w2)


def _sc_gather2(xl, xr, srcp, dstp):
    B = 40
    epw = E // NW

    @functools.partial(
        pl.kernel,
        out_type=[jax.ShapeDtypeStruct((E, HC), jnp.float32),
                  jax.ShapeDtypeStruct((E, HC), jnp.float32)],
        mesh=_mesh(),
        scratch_types=[
            pltpu.VMEM((B,), jnp.int32),
            pltpu.VMEM((B,), jnp.int32),
            pltpu.VMEM((B, HC), jnp.float32),
            pltpu.VMEM((B, HC), jnp.float32),
            pltpu.SemaphoreType.DMA,
            pltpu.SemaphoreType.DMA,
        ],
    )
    def gk(xl_hbm, xr_hbm, si_hbm, di_hbm, gl_hbm, gr_hbm,
           si_v, di_v, gl_v, gr_v, sem1, sem2):
        wid = lax.axis_index("s") * NC + lax.axis_index("c")
        base = wid * epw

        @pl.loop(0, epw, step=B)
        def _(off):
            b0 = base + off
            pltpu.sync_copy(si_hbm.at[pl.ds(b0, B)], si_v)
            pltpu.sync_copy(di_hbm.at[pl.ds(b0, B)], di_v)
            cl = pltpu.async_copy(xl_hbm.at[si_v], gl_v, sem1)
            cr = pltpu.async_copy(xr_hbm.at[di_v], gr_v, sem2)
            cl.wait()
            cr.wait()
            pltpu.sync_copy(gl_v, gl_hbm.at[pl.ds(b0, B)])
            pltpu.sync_copy(gr_v, gr_hbm.at[pl.ds(b0, B)])

    return gk(xl, xr, srcp, dstp)


def _tc_edge(gl, gr, ea, we, attf):
    BE = 640

    def body(gl_ref, gr_ref, ea_ref, we_ref, att_ref, con_ref, ex_ref):
        hh = lax.broadcasted_iota(jnp.int32, (HC, H), 0) // C
        jj = lax.broadcasted_iota(jnp.int32, (HC, H), 1)
        S = jnp.where(hh == jj, 1.0, 0.0).astype(jnp.float32)
        hh2 = lax.broadcasted_iota(jnp.int32, (H, HC), 0)
        jj2 = lax.broadcasted_iota(jnp.int32, (H, HC), 1) // C
        ST = jnp.where(hh2 == jj2, 1.0, 0.0).astype(jnp.float32)

        e = jnp.dot(ea_ref[...], we_ref[...], preferred_element_type=jnp.float32)
        glb = gl_ref[...]
        m = glb + gr_ref[...] + e
        m = jnp.where(m >= 0.0, m, 0.2 * m)
        p = jnp.dot(m * att_ref[...], S, preferred_element_type=jnp.float32)
        ex = jnp.exp(p)
        con_ref[...] = glb * jnp.dot(ex, ST, preferred_element_type=jnp.float32)
        ex_ref[...] = jnp.concatenate([ex, jnp.zeros_like(ex)], axis=1)

    return pl.pallas_call(
        body,
        grid=(E // BE,),
        in_specs=[
            pl.BlockSpec((BE, HC), lambda i: (i, 0)),
            pl.BlockSpec((BE, HC), lambda i: (i, 0)),
            pl.BlockSpec((BE, 16), lambda i: (i, 0)),
            pl.BlockSpec((16, HC), lambda i: (0, 0)),
            pl.BlockSpec((1, HC), lambda i: (0, 0)),
        ],
        out_specs=[
            pl.BlockSpec((BE, HC), lambda i: (i, 0)),
            pl.BlockSpec((BE, 16), lambda i: (i, 0)),
        ],
        out_shape=[jax.ShapeDtypeStruct((E, HC), jnp.float32),
                   jax.ShapeDtypeStruct((E, 16), jnp.float32)],
    )(gl, gr, ea, we, attf)


def _sc_scatter(con, exw, dstp):
    B = 80
    eps_ = E // NS
    rows = NH // NS

    @functools.partial(
        pl.kernel,
        out_type=[jax.ShapeDtypeStruct((NC, NH, HC), jnp.float32),
                  jax.ShapeDtypeStruct((NC, NH, 16), jnp.float32)],
        mesh=_mesh(),
        scratch_types=[
            pltpu.VMEM((B,), jnp.int32),
            pltpu.VMEM((B, HC), jnp.float32),
            pltpu.VMEM((B, 16), jnp.float32),
            pltpu.VMEM((128, HC), jnp.float32),
            pltpu.VMEM((128, 16), jnp.float32),
            pltpu.VMEM_SHARED((NH, HC), jnp.float32),
            pltpu.VMEM_SHARED((NH, 16), jnp.float32),
        ],
    )
    def sk(con_hbm, ex_hbm, di_hbm, num_hbm, den_hbm,
           idx_v, con_v, ex_v, z_v, zd_v, acc_s, den_s):
        c = lax.axis_index("c")
        s = lax.axis_index("s")

        @pl.loop(0, 128)
        def _(i):
            @pl.loop(0, HC, step=16)
            def _(j):
                z_v[i, pl.ds(j, 16)] = jnp.zeros((16,), jnp.float32)
            zd_v[i, pl.ds(0, 16)] = jnp.zeros((16,), jnp.float32)

        row0 = s * rows
        pltpu.sync_copy(z_v, acc_s.at[pl.ds(row0, 128)])
        pltpu.sync_copy(z_v, acc_s.at[pl.ds(row0 + 128, 128)])
        pltpu.sync_copy(z_v, acc_s.at[pl.ds(row0 + rows - 128, 128)])
        pltpu.sync_copy(zd_v, den_s.at[pl.ds(row0, 128)])
        pltpu.sync_copy(zd_v, den_s.at[pl.ds(row0 + 128, 128)])
        pltpu.sync_copy(zd_v, den_s.at[pl.ds(row0 + rows - 128, 128)])
        plsc.subcore_barrier()

        @pl.loop(0, eps_, step=B)
        def _(off):
            b0 = s * eps_ + off
            pltpu.sync_copy(di_hbm.at[pl.ds(b0, B)], idx_v)
            pltpu.sync_copy(con_hbm.at[pl.ds(b0, B)], con_v)
            pltpu.sync_copy(ex_hbm.at[pl.ds(b0, B)], ex_v)

            @pl.loop(0, B, step=16)
            def _(j):
                v = idx_v[pl.ds(j, 16)]
                lv = v - c * NH
                ok = (lv >= 0) & (lv < NSPLIT)
                idx_v[pl.ds(j, 16)] = jnp.where(ok, lv, NSPLIT)

            pltpu.sync_copy(con_v, acc_s.at[idx_v], add=True)
            pltpu.sync_copy(ex_v, den_s.at[idx_v], add=True)

        plsc.subcore_barrier()
        pltpu.sync_copy(acc_s.at[pl.ds(row0, rows)],
                        num_hbm.at[c].at[pl.ds(row0, rows)])
        pltpu.sync_copy(den_s.at[pl.ds(row0, rows)],
                        den_hbm.at[c].at[pl.ds(row0, rows)])

    return sk(con, exw, dstp)


def _tc_epilogue(num, den, b, concat):
    BR = 2504
    dout = HC if concat else C

    def body(num_ref, den_ref, b_ref, o_ref):
        hh2 = lax.broadcasted_iota(jnp.int32, (H, HC), 0)
        jj2 = lax.broadcasted_iota(jnp.int32, (H, HC), 1) // C
        ST = jnp.where(hh2 == jj2, 1.0, 0.0).astype(jnp.float32)
        den8 = den_ref[...][:, :H]
        denb = jnp.dot(den8, ST, preferred_element_type=jnp.float32)
        r = num_ref[...] / (denb + 1e-16)
        if concat:
            o = r
        else:
            ii = lax.broadcasted_iota(jnp.int32, (HC, C), 0) % C
            jj = lax.broadcasted_iota(jnp.int32, (HC, C), 1)
            SM = jnp.where(ii == jj, 1.0 / H, 0.0).astype(jnp.float32)
            o = jnp.dot(r, SM, preferred_element_type=jnp.float32)
        o_ref[...] = jnp.tanh(o + b_ref[...])

    return pl.pallas_call(
        body,
        grid=(NPAD // BR,),
        in_specs=[
            pl.BlockSpec((BR, HC), lambda i: (i, 0)),
            pl.BlockSpec((BR, 16), lambda i: (i, 0)),
            pl.BlockSpec((1, dout), lambda i: (0, 0)),
        ],
        out_specs=pl.BlockSpec((BR, dout), lambda i: (i, 0)),
        out_shape=jax.ShapeDtypeStruct((NPAD, dout), jnp.float32),
    )(num, den, b.reshape(1, dout))


def _tc_cls(h, w, b):
    BR = 2504
    ncls = w.shape[1]

    def body(h_ref, w_ref, b_ref, o_ref):
        lg = jnp.dot(h_ref[...], w_ref[...],
                     preferred_element_type=jnp.float32) + b_ref[...]
        mx = jnp.max(lg, axis=1, keepdims=True)
        sh = lg - mx
        o_ref[...] = sh - jnp.log(jnp.sum(jnp.exp(sh), axis=1, keepdims=True))

    return pl.pallas_call(
        body,
        grid=(NPAD // BR,),
        in_specs=[
            pl.BlockSpec((BR, HC), lambda i: (i, 0)),
            pl.BlockSpec((HC, ncls), lambda i: (0, 0)),
            pl.BlockSpec((1, ncls), lambda i: (0, 0)),
        ],
        out_specs=pl.BlockSpec((BR, ncls), lambda i: (i, 0)),
        out_shape=jax.ShapeDtypeStruct((NPAD, ncls), jnp.float32),
    )(h, w, b.reshape(1, ncls))


def kernel(x, edge_index, edge_attr,
           l1_Wl, l1_Wr, l1_We, l1_att, l1_b,
           l2_Wl, l2_Wr, l2_We, l2_att, l2_b,
           l3_Wl, l3_Wr, l3_We, l3_att, l3_b,
           cls_W, cls_b):
    src = edge_index[0].astype(jnp.int32)
    dst = edge_index[1].astype(jnp.int32)
    srcp = jnp.where(src >= NSPLIT, src + 8, src)
    dstp = jnp.where(dst >= NSPLIT, dst + 8, dst)
    zpad = jnp.zeros((8, x.shape[1]), jnp.float32)
    X = jnp.concatenate([x[:NSPLIT], zpad, x[NSPLIT:], zpad], axis=0)

    layers = [
        (l1_Wl, l1_Wr, l1_We, l1_att, l1_b, False),
        (l2_Wl, l2_Wr, l2_We, l2_att, l2_b, False),
        (l3_Wl, l3_Wr, l3_We, l3_att, l3_b, True),
    ]
    for Wl, Wr, We, att, b, concat in layers:
        xl, xr = _mm2(X, Wl, Wr)
        gl, gr = _sc_gather2(xl, xr, srcp, dstp)
        con, exw = _tc_edge(gl, gr, edge_attr, We, att.reshape(1, HC))
        num, den = _sc_scatter(con, exw, dstp)
        X = _tc_epilogue(num.reshape(NPAD, HC), den.reshape(NPAD, 16), b, concat)

    h = X
    outp = _tc_cls(h, cls_W, cls_b)

    def unpad(a):
        return jnp.concatenate([a[:NSPLIT], a[NH:NH + NSPLIT]], axis=0)

    return (unpad(outp), unpad(h))


# speedup vs baseline: 14.2433x; 14.2433x over previous
import functools

import jax
import jax.numpy as jnp
from jax import lax
from jax.experimental import pallas as pl
from jax.experimental.pallas import tpu as pltpu
from jax.experimental.pallas import tpu_sc as plsc

N = 10000
NSPLIT = 5000
NH = 5120
NPAD = 2 * NH
E = 160000
H = 8
C = 32
HC = H * C
NC = 2
NS = 16
NW = NC * NS


def _mesh():
    return plsc.VectorSubcoreMesh(core_axis_name="c", subcore_axis_name="s")


def _mm2(x, w1, w2):
    n, k = x.shape
    mo = w1.shape[1]
    br = 2560

    def body(x_ref, w1_ref, w2_ref, o1_ref, o2_ref):
        xb = x_ref[...]
        o1_ref[...] = jnp.dot(xb, w1_ref[...], preferred_element_type=jnp.float32)
        o2_ref[...] = jnp.dot(xb, w2_ref[...], preferred_element_type=jnp.float32)

    return pl.pallas_call(
        body,
        grid=(n // br,),
        in_specs=[
            pl.BlockSpec((br, k), lambda i: (i, 0)),
            pl.BlockSpec((k, mo), lambda i: (0, 0)),
            pl.BlockSpec((k, mo), lambda i: (0, 0)),
        ],
        out_specs=[
            pl.BlockSpec((br, mo), lambda i: (i, 0)),
            pl.BlockSpec((br, mo), lambda i: (i, 0)),
        ],
        out_shape=[jax.ShapeDtypeStruct((n, mo), jnp.float32),
                   jax.ShapeDtypeStruct((n, mo), jnp.float32)],
    )(x, w1, w2)


def _sc_gather2(xl, xr, srcp, dstp):
    B = 40
    epw = E // NW

    @functools.partial(
        pl.kernel,
        out_type=[jax.ShapeDtypeStruct((E, HC), jnp.float32),
                  jax.ShapeDtypeStruct((E, HC), jnp.float32)],
        mesh=_mesh(),
        scratch_types=[
            pltpu.VMEM((B,), jnp.int32),
            pltpu.VMEM((B,), jnp.int32),
            pltpu.VMEM((B, HC), jnp.float32),
            pltpu.VMEM((B, HC), jnp.float32),
            pltpu.SemaphoreType.DMA,
            pltpu.SemaphoreType.DMA,
        ],
    )
    def gk(xl_hbm, xr_hbm, si_hbm, di_hbm, gl_hbm, gr_hbm,
           si_v, di_v, gl_v, gr_v, sem1, sem2):
        wid = lax.axis_index("s") * NC + lax.axis_index("c")
        base = wid * epw

        @pl.loop(0, epw, step=B)
        def _(off):
            b0 = base + off
            pltpu.sync_copy(si_hbm.at[pl.ds(b0, B)], si_v)
            pltpu.sync_copy(di_hbm.at[pl.ds(b0, B)], di_v)
            cl = pltpu.async_copy(xl_hbm.at[si_v], gl_v, sem1)
            cr = pltpu.async_copy(xr_hbm.at[di_v], gr_v, sem2)
            cl.wait()
            cr.wait()
            pltpu.sync_copy(gl_v, gl_hbm.at[pl.ds(b0, B)])
            pltpu.sync_copy(gr_v, gr_hbm.at[pl.ds(b0, B)])

    return gk(xl, xr, srcp, dstp)


def _tc_edge(gl, gr, ea, we, attf):
    BE = 640

    def body(gl_ref, gr_ref, ea_ref, we_ref, att_ref, ca_ref, cb_ref, ex_ref):
        hh = lax.broadcasted_iota(jnp.int32, (HC, H), 0) // C
        jj = lax.broadcasted_iota(jnp.int32, (HC, H), 1)
        S = jnp.where(hh == jj, 1.0, 0.0).astype(jnp.float32)
        hh2 = lax.broadcasted_iota(jnp.int32, (H, HC), 0)
        jj2 = lax.broadcasted_iota(jnp.int32, (H, HC), 1) // C
        ST = jnp.where(hh2 == jj2, 1.0, 0.0).astype(jnp.float32)

        e = jnp.dot(ea_ref[...], we_ref[...], preferred_element_type=jnp.float32)
        glb = gl_ref[...]
        m = glb + gr_ref[...] + e
        m = jnp.where(m >= 0.0, m, 0.2 * m)
        p = jnp.dot(m * att_ref[...], S, preferred_element_type=jnp.float32)
        ex = jnp.exp(p)
        con = glb * jnp.dot(ex, ST, preferred_element_type=jnp.float32)
        ca_ref[...] = con[:, :128]
        cb_ref[...] = con[:, 128:]
        ex_ref[...] = jnp.concatenate(
            [ex, jnp.zeros((ex.shape[0], 128 - H), jnp.float32)], axis=1)

    return pl.pallas_call(
        body,
        grid=(E // BE,),
        in_specs=[
            pl.BlockSpec((BE, HC), lambda i: (i, 0)),
            pl.BlockSpec((BE, HC), lambda i: (i, 0)),
            pl.BlockSpec((BE, 16), lambda i: (i, 0)),
            pl.BlockSpec((16, HC), lambda i: (0, 0)),
            pl.BlockSpec((1, HC), lambda i: (0, 0)),
        ],
        out_specs=[
            pl.BlockSpec((BE, 128), lambda i: (i, 0)),
            pl.BlockSpec((BE, 128), lambda i: (i, 0)),
            pl.BlockSpec((BE, 128), lambda i: (i, 0)),
        ],
        out_shape=[jax.ShapeDtypeStruct((E, 128), jnp.float32),
                   jax.ShapeDtypeStruct((E, 128), jnp.float32),
                   jax.ShapeDtypeStruct((E, 128), jnp.float32)],
    )(gl, gr, ea, we, attf)


def _sc_scatter(arrs, dloc, eidx, zrow):
    B = 80
    eps_ = E // NS
    rows = NH // NS
    na = len(arrs)

    @functools.partial(
        pl.kernel,
        out_type=[jax.ShapeDtypeStruct((NC, NH, 128), jnp.float32)] * na,
        mesh=_mesh(),
        scratch_types=(
            [pltpu.VMEM((B,), jnp.int32),
             pltpu.VMEM((B,), jnp.int32)]
            + [pltpu.VMEM((B, 128), jnp.float32)] * na
            + [pltpu.VMEM_SHARED((NH, 128), jnp.float32)] * na
            + [pltpu.SemaphoreType.DMA] * na
        ),
    )
    def sk(*refs):
        src_hbm = refs[:na]
        di_hbm, ei_hbm, z_hbm = refs[na:na + 3]
        out_hbm = refs[na + 3:2 * na + 3]
        idx_v, eix_v = refs[2 * na + 3:2 * na + 5]
        val_v = refs[2 * na + 5:3 * na + 5]
        acc_s = refs[3 * na + 5:4 * na + 5]
        sems = refs[4 * na + 5:]
        c = lax.axis_index("c")
        s = lax.axis_index("s")

        row0 = s * rows
        for k in range(na):
            pltpu.sync_copy(z_hbm, acc_s[k].at[pl.ds(row0, rows)])
        plsc.subcore_barrier()

        @pl.loop(0, eps_, step=B)
        def _(off):
            b0 = s * eps_ + off
            pltpu.sync_copy(di_hbm.at[pl.ds(c * E + b0, B)], idx_v)
            pltpu.sync_copy(ei_hbm.at[pl.ds(b0, B)], eix_v)
            cps = [pltpu.async_copy(src_hbm[k].at[eix_v], val_v[k], sems[k])
                   for k in range(na)]
            for cp in cps:
                cp.wait()
            for k in range(na):
                pltpu.sync_copy(val_v[k], acc_s[k].at[idx_v], add=True)

        plsc.subcore_barrier()
        for k in range(na):
            pltpu.sync_copy(acc_s[k].at[pl.ds(row0, rows)],
                            out_hbm[k].at[c, pl.ds(row0, rows)])

    out = sk(*arrs, dloc, eidx, zrow)
    return list(out) if isinstance(out, (list, tuple)) else [out]


def _tc_epilogue(numA, numB, den, b, concat):
    BR = 2560
    dout = HC if concat else C

    def body(numA_ref, numB_ref, den_ref, b_ref, o_ref):
        hh2 = lax.broadcasted_iota(jnp.int32, (H, HC), 0)
        jj2 = lax.broadcasted_iota(jnp.int32, (H, HC), 1) // C
        ST = jnp.where(hh2 == jj2, 1.0, 0.0).astype(jnp.float32)
        den8 = den_ref[...][:, :H]
        denb = jnp.dot(den8, ST, preferred_element_type=jnp.float32)
        num = jnp.concatenate([numA_ref[...], numB_ref[...]], axis=1)
        r = num / (denb + 1e-16)
        if concat:
            o = r
        else:
            ii = lax.broadcasted_iota(jnp.int32, (HC, C), 0) % C
            jj = lax.broadcasted_iota(jnp.int32, (HC, C), 1)
            SM = jnp.where(ii == jj, 1.0 / H, 0.0).astype(jnp.float32)
            o = jnp.dot(r, SM, preferred_element_type=jnp.float32)
        o_ref[...] = jnp.tanh(o + b_ref[...])

    return pl.pallas_call(
        body,
        grid=(NPAD // BR,),
        in_specs=[
            pl.BlockSpec((BR, 128), lambda i: (i, 0)),
            pl.BlockSpec((BR, 128), lambda i: (i, 0)),
            pl.BlockSpec((BR, 128), lambda i: (i, 0)),
            pl.BlockSpec((1, dout), lambda i: (0, 0)),
        ],
        out_specs=pl.BlockSpec((BR, dout), lambda i: (i, 0)),
        out_shape=jax.ShapeDtypeStruct((NPAD, dout), jnp.float32),
    )(numA, numB, den, b.reshape(1, dout))


def _tc_cls(h, w, b):
    BR = 2560
    ncls = w.shape[1]

    def body(h_ref, w_ref, b_ref, o_ref):
        lg = jnp.dot(h_ref[...], w_ref[...],
                     preferred_element_type=jnp.float32) + b_ref[...]
        mx = jnp.max(lg, axis=1, keepdims=True)
        sh = lg - mx
        o_ref[...] = sh - jnp.log(jnp.sum(jnp.exp(sh), axis=1, keepdims=True))

    return pl.pallas_call(
        body,
        grid=(NPAD // BR,),
        in_specs=[
            pl.BlockSpec((BR, HC), lambda i: (i, 0)),
            pl.BlockSpec((HC, ncls), lambda i: (0, 0)),
            pl.BlockSpec((1, ncls), lambda i: (0, 0)),
        ],
        out_specs=pl.BlockSpec((BR, ncls), lambda i: (i, 0)),
        out_shape=jax.ShapeDtypeStruct((NPAD, ncls), jnp.float32),
    )(h, w, b.reshape(1, ncls))


def kernel(x, edge_index, edge_attr,
           l1_Wl, l1_Wr, l1_We, l1_att, l1_b,
           l2_Wl, l2_Wr, l2_We, l2_att, l2_b,
           l3_Wl, l3_Wr, l3_We, l3_att, l3_b,
           cls_W, cls_b):
    src = edge_index[0].astype(jnp.int32)
    dst = edge_index[1].astype(jnp.int32)
    srcp = jnp.where(src >= NSPLIT, src + (NH - NSPLIT), src)
    dstp = jnp.where(dst >= NSPLIT, dst + (NH - NSPLIT), dst)
    dloc0 = jnp.where(dst < NSPLIT, dst, NSPLIT)
    dloc1 = jnp.where(dst >= NSPLIT, dst - NSPLIT, NSPLIT)
    dloc = jnp.concatenate([dloc0, dloc1])
    eidx = jnp.arange(E, dtype=jnp.int32)
    zpad = jnp.zeros((NH - NSPLIT, x.shape[1]), jnp.float32)
    X = jnp.concatenate([x[:NSPLIT], zpad, x[NSPLIT:], zpad], axis=0)
    zrow = jnp.zeros((NH // NS, 128), jnp.float32)

    layers = [
        (l1_Wl, l1_Wr, l1_We, l1_att, l1_b, False),
        (l2_Wl, l2_Wr, l2_We, l2_att, l2_b, False),
        (l3_Wl, l3_Wr, l3_We, l3_att, l3_b, True),
    ]
    for Wl, Wr, We, att, b, concat in layers:
        xl, xr = _mm2(X, Wl, Wr)
        gl, gr = _sc_gather2(xl, xr, srcp, dstp)
        conA, conB, exw = _tc_edge(gl, gr, edge_attr, We, att.reshape(1, HC))
        numA, den = _sc_scatter([conA, exw], dloc, eidx, zrow)
        numB, = _sc_scatter([conB], dloc, eidx, zrow)
        X = _tc_epilogue(numA.reshape(NPAD, 128), numB.reshape(NPAD, 128),
                         den.reshape(NPAD, 128), b, concat)

    h = X
    outp = _tc_cls(h, cls_W, cls_b)

    def unpad(a):
        return jnp.concatenate([a[:NSPLIT], a[NH:NH + NSPLIT]], axis=0)

    return (unpad(outp), unpad(h))


# speedup vs baseline: 15.2125x; 1.0680x over previous
import functools

import jax
import jax.numpy as jnp
from jax import lax
from jax.experimental import pallas as pl
from jax.experimental.pallas import tpu as pltpu
from jax.experimental.pallas import tpu_sc as plsc

N = 10000
NSPLIT = 5000
NH = 5120
NPAD = 2 * NH
E = 160000
H = 8
C = 32
HC = H * C
NC = 2
NS = 16
NW = NC * NS


def _mesh():
    return plsc.VectorSubcoreMesh(core_axis_name="c", subcore_axis_name="s")


def _mm2(x, w1, w2):
    n, k = x.shape
    mo = w1.shape[1]
    br = 2560

    def body(x_ref, w1_ref, w2_ref, o1_ref, o2_ref):
        xb = x_ref[...]
        o1_ref[...] = jnp.dot(xb, w1_ref[...], preferred_element_type=jnp.float32)
        o2_ref[...] = jnp.dot(xb, w2_ref[...], preferred_element_type=jnp.float32)

    return pl.pallas_call(
        body,
        grid=(n // br,),
        in_specs=[
            pl.BlockSpec((br, k), lambda i: (i, 0)),
            pl.BlockSpec((k, mo), lambda i: (0, 0)),
            pl.BlockSpec((k, mo), lambda i: (0, 0)),
        ],
        out_specs=[
            pl.BlockSpec((br, mo), lambda i: (i, 0)),
            pl.BlockSpec((br, mo), lambda i: (i, 0)),
        ],
        out_shape=[jax.ShapeDtypeStruct((n, mo), jnp.float32),
                   jax.ShapeDtypeStruct((n, mo), jnp.float32)],
    )(x, w1, w2)


def _sc_gather2(xl, xr, srcp, dstp):
    B = 40
    epw = E // NW

    @functools.partial(
        pl.kernel,
        out_type=[jax.ShapeDtypeStruct((E, HC), jnp.float32),
                  jax.ShapeDtypeStruct((E, HC), jnp.float32)],
        mesh=_mesh(),
        scratch_types=[
            pltpu.VMEM((B,), jnp.int32),
            pltpu.VMEM((B,), jnp.int32),
            pltpu.VMEM((B, HC), jnp.float32),
            pltpu.VMEM((B, HC), jnp.float32),
            pltpu.SemaphoreType.DMA,
            pltpu.SemaphoreType.DMA,
        ],
    )
    def gk(xl_hbm, xr_hbm, si_hbm, di_hbm, gl_hbm, gr_hbm,
           si_v, di_v, gl_v, gr_v, sem1, sem2):
        wid = lax.axis_index("s") * NC + lax.axis_index("c")
        base = wid * epw

        @pl.loop(0, epw, step=B)
        def _(off):
            b0 = base + off
            pltpu.sync_copy(si_hbm.at[pl.ds(b0, B)], si_v)
            pltpu.sync_copy(di_hbm.at[pl.ds(b0, B)], di_v)
            cl = pltpu.async_copy(xl_hbm.at[si_v], gl_v, sem1)
            cr = pltpu.async_copy(xr_hbm.at[di_v], gr_v, sem2)
            cl.wait()
            cr.wait()
            pltpu.sync_copy(gl_v, gl_hbm.at[pl.ds(b0, B)])
            pltpu.sync_copy(gr_v, gr_hbm.at[pl.ds(b0, B)])

    return gk(xl, xr, srcp, dstp)


def _tc_edge(gl, gr, ea, we, attf, offg):
    BE = 640

    def body(gl_ref, gr_ref, ea_ref, we_ref, att_ref, og_ref,
             ca_ref, cb_ref, ex_ref):
        hh = lax.broadcasted_iota(jnp.int32, (HC, H), 0) // C
        jj = lax.broadcasted_iota(jnp.int32, (HC, H), 1)
        S = jnp.where(hh == jj, 1.0, 0.0).astype(jnp.float32)
        hh2 = lax.broadcasted_iota(jnp.int32, (H, HC), 0)
        jj2 = lax.broadcasted_iota(jnp.int32, (H, HC), 1) // C
        ST = jnp.where(hh2 == jj2, 1.0, 0.0).astype(jnp.float32)

        e = jnp.dot(ea_ref[...], we_ref[...], preferred_element_type=jnp.float32)
        glb = gl_ref[...]
        m = glb + gr_ref[...] + e
        m = jnp.where(m >= 0.0, m, 0.2 * m)
        p = jnp.dot(m * att_ref[...], S, preferred_element_type=jnp.float32)
        ex = jnp.exp(p)
        con = glb * jnp.dot(ex, ST, preferred_element_type=jnp.float32)
        ca_ref[...] = con[:, :128]
        cb_ref[...] = con[:, 128:]
        jt = lax.broadcasted_iota(jnp.int32, (H, 128), 0)
        lt = lax.broadcasted_iota(jnp.int32, (H, 128), 1) % H
        T8 = jnp.where(lt == jt, 1.0, 0.0).astype(jnp.float32)
        tiled = jnp.dot(ex, T8, preferred_element_type=jnp.float32)
        grp = lax.broadcasted_iota(jnp.int32, (ex.shape[0], 128), 1) // H
        ex_ref[...] = jnp.where(grp == og_ref[...], tiled, 0.0)

    return pl.pallas_call(
        body,
        grid=(E // BE,),
        in_specs=[
            pl.BlockSpec((BE, HC), lambda i: (i, 0)),
            pl.BlockSpec((BE, HC), lambda i: (i, 0)),
            pl.BlockSpec((BE, 16), lambda i: (i, 0)),
            pl.BlockSpec((16, HC), lambda i: (0, 0)),
            pl.BlockSpec((1, HC), lambda i: (0, 0)),
            pl.BlockSpec((BE, 1), lambda i: (i, 0)),
        ],
        out_specs=[
            pl.BlockSpec((BE, 128), lambda i: (i, 0)),
            pl.BlockSpec((BE, 128), lambda i: (i, 0)),
            pl.BlockSpec((BE, 128), lambda i: (i, 0)),
        ],
        out_shape=[jax.ShapeDtypeStruct((E, 128), jnp.float32),
                   jax.ShapeDtypeStruct((E, 128), jnp.float32),
                   jax.ShapeDtypeStruct((E, 128), jnp.float32)],
    )(gl, gr, ea, we, attf, offg)


DROWS = NH // 16


def _sc_scatter(conA, conB, exw, dloc, dloc16, eidx, zrow):
    B = 80
    eps_ = E // NS
    rows = NH // NS

    @functools.partial(
        pl.kernel,
        out_type=[jax.ShapeDtypeStruct((NC, NH, 128), jnp.float32),
                  jax.ShapeDtypeStruct((NC, NH, 128), jnp.float32),
                  jax.ShapeDtypeStruct((NC, DROWS, 128), jnp.float32)],
        mesh=_mesh(),
        scratch_types=[
            pltpu.VMEM((B,), jnp.int32),
            pltpu.VMEM((B,), jnp.int32),
            pltpu.VMEM((B,), jnp.int32),
            pltpu.VMEM((B, 128), jnp.float32),
            pltpu.VMEM((B, 128), jnp.float32),
            pltpu.VMEM((B, 128), jnp.float32),
            pltpu.VMEM_SHARED((NH, 128), jnp.float32),
            pltpu.VMEM_SHARED((NH, 128), jnp.float32),
            pltpu.VMEM_SHARED((DROWS, 128), jnp.float32),
            pltpu.SemaphoreType.DMA,
            pltpu.SemaphoreType.DMA,
            pltpu.SemaphoreType.DMA,
        ],
    )
    def sk(ca_hbm, cb_hbm, ex_hbm, di_hbm, d16_hbm, ei_hbm, z_hbm,
           oa_hbm, ob_hbm, od_hbm,
           idx_v, i16_v, eix_v, va_v, vb_v, vd_v,
           accA, accB, accD, semA, semB, semD):
        c = lax.axis_index("c")
        s = lax.axis_index("s")

        row0 = s * rows
        pltpu.sync_copy(z_hbm, accA.at[pl.ds(row0, rows)])
        pltpu.sync_copy(z_hbm, accB.at[pl.ds(row0, rows)])
        @pl.when(s < 10)
        def _():
            pltpu.sync_copy(z_hbm.at[pl.ds(0, DROWS // 10)],
                            accD.at[pl.ds(s * (DROWS // 10), DROWS // 10)])
        plsc.subcore_barrier()

        @pl.loop(0, eps_, step=B)
        def _(off):
            b0 = s * eps_ + off
            pltpu.sync_copy(di_hbm.at[pl.ds(c * E + b0, B)], idx_v)
            pltpu.sync_copy(d16_hbm.at[pl.ds(c * E + b0, B)], i16_v)
            pltpu.sync_copy(ei_hbm.at[pl.ds(b0, B)], eix_v)
            cpa = pltpu.async_copy(ca_hbm.at[eix_v], va_v, semA)
            cpb = pltpu.async_copy(cb_hbm.at[eix_v], vb_v, semB)
            cpd = pltpu.async_copy(ex_hbm.at[eix_v], vd_v, semD)
            cpa.wait()
            cpb.wait()
            cpd.wait()
            pltpu.sync_copy(va_v, accA.at[idx_v], add=True)
            pltpu.sync_copy(vb_v, accB.at[idx_v], add=True)
            pltpu.sync_copy(vd_v, accD.at[i16_v], add=True)

        plsc.subcore_barrier()
        pltpu.sync_copy(accA.at[pl.ds(row0, rows)],
                        oa_hbm.at[c, pl.ds(row0, rows)])
        pltpu.sync_copy(accB.at[pl.ds(row0, rows)],
                        ob_hbm.at[c, pl.ds(row0, rows)])
        @pl.when(s < 10)
        def _():
            pltpu.sync_copy(accD.at[pl.ds(s * (DROWS // 10), DROWS // 10)],
                            od_hbm.at[c, pl.ds(s * (DROWS // 10), DROWS // 10)])

    return sk(conA, conB, exw, dloc, dloc16, eidx, zrow)


def _tc_epilogue(numA, numB, den, b, concat):
    BR = 2560
    dout = HC if concat else C

    def body(numA_ref, numB_ref, den_ref, b_ref, o_ref):
        ii = lax.broadcasted_iota(jnp.int32, (BR, BR // 16), 0) // 16
        jj = lax.broadcasted_iota(jnp.int32, (BR, BR // 16), 1)
        G = jnp.where(ii == jj, 1.0, 0.0).astype(jnp.float32)
        den_dup = jnp.dot(G, den_ref[...], preferred_element_type=jnp.float32)
        rm = lax.broadcasted_iota(jnp.int32, (BR, 128), 0) % 16
        gl_ = lax.broadcasted_iota(jnp.int32, (BR, 128), 1) // H
        tmp = jnp.where(gl_ == rm, den_dup, 0.0)
        ll = lax.broadcasted_iota(jnp.int32, (128, HC), 0) % H
        hh = lax.broadcasted_iota(jnp.int32, (128, HC), 1) // C
        RST = jnp.where(ll == hh, 1.0, 0.0).astype(jnp.float32)
        denb = jnp.dot(tmp, RST, preferred_element_type=jnp.float32)
        num = jnp.concatenate([numA_ref[...], numB_ref[...]], axis=1)
        r = num / (denb + 1e-16)
        if concat:
            o = r
        else:
            ii = lax.broadcasted_iota(jnp.int32, (HC, C), 0) % C
            jj = lax.broadcasted_iota(jnp.int32, (HC, C), 1)
            SM = jnp.where(ii == jj, 1.0 / H, 0.0).astype(jnp.float32)
            o = jnp.dot(r, SM, preferred_element_type=jnp.float32)
        o_ref[...] = jnp.tanh(o + b_ref[...])

    return pl.pallas_call(
        body,
        grid=(NPAD // BR,),
        in_specs=[
            pl.BlockSpec((BR, 128), lambda i: (i, 0)),
            pl.BlockSpec((BR, 128), lambda i: (i, 0)),
            pl.BlockSpec((BR // 16, 128), lambda i: (i, 0)),
            pl.BlockSpec((1, dout), lambda i: (0, 0)),
        ],
        out_specs=pl.BlockSpec((BR, dout), lambda i: (i, 0)),
        out_shape=jax.ShapeDtypeStruct((NPAD, dout), jnp.float32),
    )(numA, numB, den, b.reshape(1, dout))


def _tc_cls(h, w, b):
    BR = 2560
    ncls = w.shape[1]

    def body(h_ref, w_ref, b_ref, o_ref):
        lg = jnp.dot(h_ref[...], w_ref[...],
                     preferred_element_type=jnp.float32) + b_ref[...]
        mx = jnp.max(lg, axis=1, keepdims=True)
        sh = lg - mx
        o_ref[...] = sh - jnp.log(jnp.sum(jnp.exp(sh), axis=1, keepdims=True))

    return pl.pallas_call(
        body,
        grid=(NPAD // BR,),
        in_specs=[
            pl.BlockSpec((BR, HC), lambda i: (i, 0)),
            pl.BlockSpec((HC, ncls), lambda i: (0, 0)),
            pl.BlockSpec((1, ncls), lambda i: (0, 0)),
        ],
        out_specs=pl.BlockSpec((BR, ncls), lambda i: (i, 0)),
        out_shape=jax.ShapeDtypeStruct((NPAD, ncls), jnp.float32),
    )(h, w, b.reshape(1, ncls))


def kernel(x, edge_index, edge_attr,
           l1_Wl, l1_Wr, l1_We, l1_att, l1_b,
           l2_Wl, l2_Wr, l2_We, l2_att, l2_b,
           l3_Wl, l3_Wr, l3_We, l3_att, l3_b,
           cls_W, cls_b):
    src = edge_index[0].astype(jnp.int32)
    dst = edge_index[1].astype(jnp.int32)
    srcp = jnp.where(src >= NSPLIT, src + (NH - NSPLIT), src)
    dstp = jnp.where(dst >= NSPLIT, dst + (NH - NSPLIT), dst)
    dloc0 = jnp.where(dst < NSPLIT, dst, NSPLIT)
    dloc1 = jnp.where(dst >= NSPLIT, dst - NSPLIT, NSPLIT)
    dloc = jnp.concatenate([dloc0, dloc1])
    d16_0 = jnp.where(dst < NSPLIT, dst // 16, DROWS - 1)
    d16_1 = jnp.where(dst >= NSPLIT, (dst - NSPLIT) // 16, DROWS - 1)
    dloc16 = jnp.concatenate([d16_0, d16_1])
    offg = (jnp.where(dst < NSPLIT, dst, dst - NSPLIT) % 16
            ).astype(jnp.int32).reshape(E, 1)
    eidx = jnp.arange(E, dtype=jnp.int32)
    zpad = jnp.zeros((NH - NSPLIT, x.shape[1]), jnp.float32)
    X = jnp.concatenate([x[:NSPLIT], zpad, x[NSPLIT:], zpad], axis=0)
    zrow = jnp.zeros((NH // NS, 128), jnp.float32)

    layers = [
        (l1_Wl, l1_Wr, l1_We, l1_att, l1_b, False),
        (l2_Wl, l2_Wr, l2_We, l2_att, l2_b, False),
        (l3_Wl, l3_Wr, l3_We, l3_att, l3_b, True),
    ]
    for Wl, Wr, We, att, b, concat in layers:
        xl, xr = _mm2(X, Wl, Wr)
        gl, gr = _sc_gather2(xl, xr, srcp, dstp)
        conA, conB, exw = _tc_edge(gl, gr, edge_attr, We,
                                   att.reshape(1, HC), offg)
        numA, numB, den = _sc_scatter(conA, conB, exw, dloc, dloc16,
                                      eidx, zrow)
        X = _tc_epilogue(numA.reshape(NPAD, 128), numB.reshape(NPAD, 128),
                         den.reshape(NC * DROWS, 128), b, concat)

    h = X
    outp = _tc_cls(h, cls_W, cls_b)

    def unpad(a):
        return jnp.concatenate([a[:NSPLIT], a[NH:NH + NSPLIT]], axis=0)

    return (unpad(outp), unpad(h))
